# baseline (speedup 1.0000x reference)
import jax
import jax.numpy as jnp
from jax.experimental import pallas as pl
from jax.experimental.pallas import tpu as pltpu

_B = 800000
_D = 64
_R = _B // 2
_BLK = 8192


def _dot_rows_kernel(x_ref, o_ref):
    p = x_ref[0] * x_ref[1]
    s0 = jnp.sum(p[:, :_D], axis=1, keepdims=True)
    s1 = jnp.sum(p[:, _D:], axis=1, keepdims=True)
    o_ref[...] = jnp.concatenate([s0, s1], axis=1)


def kernel(inputs):
    flat = inputs.reshape(2, _R, 2 * _D)
    out2 = pl.pallas_call(
        _dot_rows_kernel,
        grid=(pl.cdiv(_R, _BLK),),
        in_specs=[pl.BlockSpec((2, _BLK, 2 * _D), lambda i: (0, i, 0))],
        out_specs=pl.BlockSpec((_BLK, 2), lambda i: (i, 0)),
        out_shape=jax.ShapeDtypeStruct((_R, 2), jnp.float32),
        compiler_params=pltpu.CompilerParams(
            dimension_semantics=("parallel",),
        ),
    )(flat)
    return out2.reshape(_B)


# speedup vs baseline: 10.2136x; 10.2136x over previous
import jax
import jax.numpy as jnp
from jax.experimental import pallas as pl
from jax.experimental.pallas import tpu as pltpu

_B = 800000
_D = 64
_BLK = 16384


def _dot_rows_kernel(x_ref, o_ref):
    p = x_ref[0] * x_ref[1]
    o_ref[...] = jnp.sum(p, axis=0)


def kernel(inputs):
    t = jnp.transpose(inputs, (0, 2, 1))
    out = pl.pallas_call(
        _dot_rows_kernel,
        grid=(pl.cdiv(_B, _BLK),),
        in_specs=[pl.BlockSpec((2, _D, _BLK), lambda i: (0, 0, i))],
        out_specs=pl.BlockSpec((_BLK,), lambda i: (i,)),
        out_shape=jax.ShapeDtypeStruct((_B,), jnp.float32),
        compiler_params=pltpu.CompilerParams(
            dimension_semantics=("parallel",),
        ),
    )(t)
    return out
